# baseline (speedup 1.0000x reference)
import jax
import jax.numpy as jnp
from jax.experimental import pallas as pl
from jax.experimental.pallas import tpu as pltpu


def _rup8(n):
    return (n + 7) & ~7


def _slab_offsets(d_in, h):
    o_w1 = 0
    o_b1 = _rup8(o_w1 + d_in)
    o_wih = _rup8(o_b1 + 1)
    o_whh = o_wih + _rup8(h)
    o_bl = o_whh + _rup8(h)
    main_rows = _rup8(o_bl + 1)
    e_w2 = 0
    e_b2 = _rup8(e_w2 + h)
    e_w3 = _rup8(e_b2 + 1)
    e_b3 = _rup8(e_w3 + 1)
    epi_rows = _rup8(e_b3 + 1)
    return dict(o_w1=o_w1, o_b1=o_b1, o_wih=o_wih, o_whh=o_whh, o_bl=o_bl,
                main_rows=main_rows, e_w2=e_w2, e_b2=e_b2, e_w3=e_w3,
                e_b3=e_b3, epi_rows=epi_rows)


def _lstm_block_kernel(x_ref, hc0_ref, epi_ref, main_hbm,
                       q_ref, hcT_ref,
                       w1_buf, wih_buf, whh_buf,
                       sem1, sem2, sem3):
    T, Bc, d_in = x_ref.shape
    H = hc0_ref.shape[2]
    L = _slab_offsets(d_in, H)

    n1 = L["o_wih"]
    n2 = L["o_whh"] - L["o_wih"]
    n3 = L["main_rows"] - L["o_whh"]
    cp1 = pltpu.make_async_copy(main_hbm.at[pl.ds(0, n1)], w1_buf, sem1)
    cp2 = pltpu.make_async_copy(main_hbm.at[pl.ds(L["o_wih"], n2)], wih_buf, sem2)
    cp3 = pltpu.make_async_copy(main_hbm.at[pl.ds(L["o_whh"], n3)], whh_buf, sem3)
    cp1.start()
    cp2.start()
    cp3.start()

    x = x_ref[...].reshape(T * Bc, d_in)

    cp1.wait()
    w1 = w1_buf[0:d_in, 0:H]
    b1 = w1_buf[L["o_b1"]:L["o_b1"] + 1, 0:H]
    a1 = jnp.maximum(
        jnp.dot(x, w1, preferred_element_type=jnp.float32) + b1, 0.0)

    cp2.wait()
    pre = jnp.dot(a1, wih_buf[0:H, :], preferred_element_type=jnp.float32)

    cp3.wait()
    w_hh = whh_buf[0:H, :]
    b_l = whh_buf[L["o_bl"] - L["o_whh"]:L["o_bl"] - L["o_whh"] + 1, :]
    pre = pre + b_l

    h = hc0_ref[0]
    c = hc0_ref[1]
    hs_steps = []
    for t in range(T):
        gt = pre[t * Bc:(t + 1) * Bc, :] + jnp.dot(
            h, w_hh, preferred_element_type=jnp.float32)
        s_if = jax.nn.sigmoid(gt[:, 0:2 * H])
        o_g = jax.nn.sigmoid(gt[:, 3 * H:4 * H])
        g_g = jnp.tanh(gt[:, 2 * H:3 * H])
        c = s_if[:, H:2 * H] * c + s_if[:, 0:H] * g_g
        h = o_g * jnp.tanh(c)
        hs_steps.append(h)

    hcT_ref[0] = h
    hcT_ref[1] = c

    hs = jnp.concatenate(hs_steps, axis=0)
    w2 = epi_ref[0:H, 0:H]
    b2 = epi_ref[L["e_b2"]:L["e_b2"] + 1, 0:H]
    w3r = epi_ref[L["e_w3"]:L["e_w3"] + 1, 0:H]
    b3 = epi_ref[L["e_b3"]:L["e_b3"] + 1, 0:1]
    a2 = jnp.maximum(
        jnp.dot(hs, w2, preferred_element_type=jnp.float32) + b2, 0.0)
    q_ref[0] = jnp.dot(w3r, a2.T, preferred_element_type=jnp.float32) + b3


def kernel(main, epi, state, action, last_action, h0, c0):
    B, T, _ = state.shape
    H = h0.shape[-1]
    d_in = state.shape[-1] + action.shape[-1] + last_action.shape[-1]
    L = _slab_offsets(d_in, H)
    G = 4 * H
    NC = 2
    Bc = B // NC

    x = jnp.concatenate([state, action, last_action], axis=-1)
    x = jnp.transpose(x, (1, 0, 2)).astype(jnp.float32)
    hc0 = jnp.concatenate([h0, c0], axis=0).astype(jnp.float32)

    flops = 2 * T * B * (d_in * H + 2 * H * 4 * H + H * H + H)
    nbytes = 4 * (x.size + hc0.size + NC * main.size
                  + NC * L["epi_rows"] * H + T * B + 2 * B * H)
    cost = pl.CostEstimate(flops=flops,
                           transcendentals=3 * T * B * H,
                           bytes_accessed=nbytes)

    n1 = L["o_wih"]
    n2 = L["o_whh"] - L["o_wih"]
    n3 = L["main_rows"] - L["o_whh"]

    q_parts, hcT = pl.pallas_call(
        _lstm_block_kernel,
        out_shape=(
            jax.ShapeDtypeStruct((NC, 1, T * Bc), jnp.float32),
            jax.ShapeDtypeStruct((2, B, H), jnp.float32),
        ),
        grid_spec=pltpu.PrefetchScalarGridSpec(
            num_scalar_prefetch=0,
            grid=(NC,),
            in_specs=[
                pl.BlockSpec((T, Bc, d_in), lambda i: (0, i, 0)),
                pl.BlockSpec((2, Bc, H), lambda i: (0, i, 0)),
                pl.BlockSpec((L["epi_rows"], H), lambda i: (0, 0)),
                pl.BlockSpec(memory_space=pl.ANY),
            ],
            out_specs=[
                pl.BlockSpec((1, 1, T * Bc), lambda i: (i, 0, 0)),
                pl.BlockSpec((2, Bc, H), lambda i: (0, i, 0)),
            ],
            scratch_shapes=[
                pltpu.VMEM((n1, G), jnp.float32),
                pltpu.VMEM((n2, G), jnp.float32),
                pltpu.VMEM((n3, G), jnp.float32),
                pltpu.SemaphoreType.DMA,
                pltpu.SemaphoreType.DMA,
                pltpu.SemaphoreType.DMA,
            ],
        ),
        compiler_params=pltpu.CompilerParams(
            dimension_semantics=("parallel",),
        ),
        cost_estimate=cost,
    )(x, hc0, epi, main)

    q = q_parts.reshape(NC, T, Bc).transpose(0, 2, 1).reshape(B, T)[..., None]
    return q, (hcT[0][None], hcT[1][None])


# speedup vs baseline: 1.6005x; 1.6005x over previous
import jax
import jax.numpy as jnp
from jax.experimental import pallas as pl
from jax.experimental.pallas import tpu as pltpu


def _rup8(n):
    return (n + 7) & ~7


def _slab_offsets(d_in, h):
    o_w1 = 0
    o_b1 = _rup8(o_w1 + d_in)
    o_wih = _rup8(o_b1 + 1)
    o_whh = o_wih + _rup8(h)
    o_bl = o_whh + _rup8(h)
    main_rows = _rup8(o_bl + 1)
    e_w2 = 0
    e_b2 = _rup8(e_w2 + h)
    e_w3 = _rup8(e_b2 + 1)
    e_b3 = _rup8(e_w3 + 1)
    epi_rows = _rup8(e_b3 + 1)
    return dict(o_w1=o_w1, o_b1=o_b1, o_wih=o_wih, o_whh=o_whh, o_bl=o_bl,
                main_rows=main_rows, e_w2=e_w2, e_b2=e_b2, e_w3=e_w3,
                e_b3=e_b3, epi_rows=epi_rows)


def _fused_kernel(x_ref, h0_ref, c0_ref, epi_ref, main_hbm,
                  q_ref, hn_ref, cn_ref,
                  w1_buf, wih_buf, whh_buf, wih_bf,
                  sem1, sem2, sem3):
    TB, d_in = x_ref.shape
    H = h0_ref.shape[2]
    B = h0_ref.shape[1]
    T = TB // B
    L = _slab_offsets(d_in, H)

    cp1 = pltpu.make_async_copy(
        main_hbm.at[pl.ds(0, L["o_wih"]), pl.ds(0, H)], w1_buf, sem1)
    cp2 = pltpu.make_async_copy(
        main_hbm.at[pl.ds(L["o_wih"], H)], wih_buf, sem2)
    cp3 = pltpu.make_async_copy(
        main_hbm.at[pl.ds(L["o_whh"], L["main_rows"] - L["o_whh"])],
        whh_buf, sem3)
    cp1.start()
    cp2.start()
    cp3.start()

    cp1.wait()
    w1b = w1_buf[0:d_in, :].astype(jnp.bfloat16)
    b1 = w1_buf[L["o_b1"]:L["o_b1"] + 1, :]
    a1 = jnp.maximum(
        jnp.dot(x_ref[...], w1b, preferred_element_type=jnp.float32) + b1,
        0.0).astype(jnp.bfloat16)

    cp2.wait()
    wih_bf[...] = wih_buf[...].astype(jnp.bfloat16)

    cp3.wait()
    w_hh = whh_buf[0:H, :]
    b_l = whh_buf[L["o_bl"] - L["o_whh"]:L["o_bl"] - L["o_whh"] + 1, :]

    w2b = epi_ref[0:H, :].astype(jnp.bfloat16)
    b2 = epi_ref[L["e_b2"]:L["e_b2"] + 1, :]
    w3b = epi_ref[L["e_w3"]:L["e_w3"] + 1, :].astype(jnp.bfloat16)
    b3 = epi_ref[L["e_b3"]:L["e_b3"] + 1, 0:1]

    h = h0_ref[0]
    c = c0_ref[0]
    a2_steps = []
    for t in range(T):
        pre_t = jnp.dot(a1[t * B:(t + 1) * B, :], wih_bf[...],
                        preferred_element_type=jnp.float32) + b_l
        gt = pre_t + jnp.dot(h, w_hh, preferred_element_type=jnp.float32)
        s_if = jax.nn.sigmoid(gt[:, 0:2 * H])
        o_g = jax.nn.sigmoid(gt[:, 3 * H:4 * H])
        g_g = jnp.tanh(gt[:, 2 * H:3 * H])
        c = s_if[:, H:2 * H] * c + s_if[:, 0:H] * g_g
        h = o_g * jnp.tanh(c)
        a2_t = jnp.maximum(
            jnp.dot(h.astype(jnp.bfloat16), w2b,
                    preferred_element_type=jnp.float32) + b2, 0.0)
        a2_steps.append(a2_t.astype(jnp.bfloat16))

    hn_ref[0] = h
    cn_ref[0] = c

    a2 = jnp.concatenate(a2_steps, axis=0)
    q_ref[...] = jnp.dot(w3b, a2.T, preferred_element_type=jnp.float32) + b3


def kernel(main, epi, state, action, last_action, h0, c0):
    B, T, _ = state.shape
    H = h0.shape[-1]
    d_in = state.shape[-1] + action.shape[-1] + last_action.shape[-1]
    L = _slab_offsets(d_in, H)
    G = 4 * H

    x = jnp.concatenate([state, action, last_action], axis=-1)
    x = jnp.transpose(x, (1, 0, 2)).reshape(T * B, d_in).astype(jnp.bfloat16)

    flops = 2 * T * B * (d_in * H + 2 * H * 4 * H + H * H + H)
    nbytes = (2 * x.size + 4 * (2 * B * H + L["o_wih"] * H
                                + (L["main_rows"] - L["o_wih"]) * G
                                + L["epi_rows"] * H + T * B + 2 * B * H))
    cost = pl.CostEstimate(flops=flops,
                           transcendentals=3 * T * B * H,
                           bytes_accessed=nbytes)

    q_row, h_n, c_n = pl.pallas_call(
        _fused_kernel,
        out_shape=(
            jax.ShapeDtypeStruct((1, T * B), jnp.float32),
            jax.ShapeDtypeStruct((1, B, H), jnp.float32),
            jax.ShapeDtypeStruct((1, B, H), jnp.float32),
        ),
        grid_spec=pltpu.PrefetchScalarGridSpec(
            num_scalar_prefetch=0,
            grid=(1,),
            in_specs=[
                pl.BlockSpec((T * B, d_in), lambda i: (0, 0)),
                pl.BlockSpec((1, B, H), lambda i: (0, 0, 0)),
                pl.BlockSpec((1, B, H), lambda i: (0, 0, 0)),
                pl.BlockSpec((L["epi_rows"], H), lambda i: (0, 0)),
                pl.BlockSpec(memory_space=pl.ANY),
            ],
            out_specs=[
                pl.BlockSpec((1, T * B), lambda i: (0, 0)),
                pl.BlockSpec((1, B, H), lambda i: (0, 0, 0)),
                pl.BlockSpec((1, B, H), lambda i: (0, 0, 0)),
            ],
            scratch_shapes=[
                pltpu.VMEM((L["o_wih"], H), jnp.float32),
                pltpu.VMEM((H, G), jnp.float32),
                pltpu.VMEM((L["main_rows"] - L["o_whh"], G), jnp.float32),
                pltpu.VMEM((H, G), jnp.bfloat16),
                pltpu.SemaphoreType.DMA,
                pltpu.SemaphoreType.DMA,
                pltpu.SemaphoreType.DMA,
            ],
        ),
        compiler_params=pltpu.CompilerParams(
            dimension_semantics=("arbitrary",),
        ),
        cost_estimate=cost,
    )(x, h0, c0, epi, main)

    q = q_row.reshape(T, B).T[..., None]
    return q, (h_n, c_n)


# speedup vs baseline: 1.6448x; 1.0277x over previous
import jax
import jax.numpy as jnp
from jax.experimental import pallas as pl
from jax.experimental.pallas import tpu as pltpu


def _rup8(n):
    return (n + 7) & ~7


def _slab_offsets(d_in, h):
    o_w1 = 0
    o_b1 = _rup8(o_w1 + d_in)
    o_wih = _rup8(o_b1 + 1)
    o_whh = o_wih + _rup8(h)
    o_bl = o_whh + _rup8(h)
    main_rows = _rup8(o_bl + 1)
    e_w2 = 0
    e_b2 = _rup8(e_w2 + h)
    e_w3 = _rup8(e_b2 + 1)
    e_b3 = _rup8(e_w3 + 1)
    epi_rows = _rup8(e_b3 + 1)
    return dict(o_w1=o_w1, o_b1=o_b1, o_wih=o_wih, o_whh=o_whh, o_bl=o_bl,
                main_rows=main_rows, e_w2=e_w2, e_b2=e_b2, e_w3=e_w3,
                e_b3=e_b3, epi_rows=epi_rows)


def _fused_kernel(x_ref, h0_ref, c0_ref, epi_ref, main_hbm,
                  q_ref, hn_ref, cn_ref,
                  w1_buf, wg_buf,
                  sem1, sem2):
    TB, d_in = x_ref.shape
    H = h0_ref.shape[2]
    B = h0_ref.shape[1]
    T = TB // B
    L = _slab_offsets(d_in, H)

    cp1 = pltpu.make_async_copy(
        main_hbm.at[pl.ds(0, L["o_wih"]), pl.ds(0, H)], w1_buf, sem1)
    cp2 = pltpu.make_async_copy(
        main_hbm.at[pl.ds(L["o_wih"], L["main_rows"] - L["o_wih"])],
        wg_buf, sem2)
    cp1.start()
    cp2.start()

    cp1.wait()
    w1b = w1_buf[0:d_in, :].astype(jnp.bfloat16)
    b1 = w1_buf[L["o_b1"]:L["o_b1"] + 1, :]
    a1 = jnp.maximum(
        jnp.dot(x_ref[...], w1b, preferred_element_type=jnp.float32) + b1,
        0.0)

    w2 = epi_ref[0:H, :]
    b2 = epi_ref[L["e_b2"]:L["e_b2"] + 1, :]
    w3r = epi_ref[L["e_w3"]:L["e_w3"] + 1, :]
    b3 = epi_ref[L["e_b3"]:L["e_b3"] + 1, 0:1]

    cp2.wait()
    wg = wg_buf[0:2 * H, :]
    b_l = wg_buf[L["o_bl"] - L["o_wih"]:L["o_bl"] - L["o_wih"] + 1, :]

    h = h0_ref[0]
    c = c0_ref[0]
    a2_steps = []
    for t in range(T):
        lhs = jnp.concatenate([a1[t * B:(t + 1) * B, :], h], axis=1)
        gt = jnp.dot(lhs, wg, preferred_element_type=jnp.float32) + b_l
        s_if = jax.nn.sigmoid(gt[:, 0:2 * H])
        o_g = jax.nn.sigmoid(gt[:, 3 * H:4 * H])
        g_g = jnp.tanh(gt[:, 2 * H:3 * H])
        c = s_if[:, H:2 * H] * c + s_if[:, 0:H] * g_g
        h = o_g * jnp.tanh(c)
        a2_t = jnp.maximum(
            jnp.dot(h, w2, preferred_element_type=jnp.float32) + b2, 0.0)
        a2_steps.append(a2_t)

    hn_ref[0] = h
    cn_ref[0] = c

    a2 = jnp.concatenate(a2_steps, axis=0)
    q_ref[...] = jnp.dot(w3r, a2.T, preferred_element_type=jnp.float32) + b3


def kernel(main, epi, state, action, last_action, h0, c0):
    B, T, _ = state.shape
    H = h0.shape[-1]
    d_in = state.shape[-1] + action.shape[-1] + last_action.shape[-1]
    L = _slab_offsets(d_in, H)
    G = 4 * H

    x = jnp.concatenate([state, action, last_action], axis=-1)
    x = jnp.transpose(x, (1, 0, 2)).reshape(T * B, d_in).astype(jnp.bfloat16)

    flops = 2 * T * B * (d_in * H + 2 * H * 4 * H + H * H + H)
    nbytes = (2 * x.size + 4 * (2 * B * H + L["o_wih"] * H
                                + (L["main_rows"] - L["o_wih"]) * G
                                + L["epi_rows"] * H + T * B + 2 * B * H))
    cost = pl.CostEstimate(flops=flops,
                           transcendentals=3 * T * B * H,
                           bytes_accessed=nbytes)

    q_row, h_n, c_n = pl.pallas_call(
        _fused_kernel,
        out_shape=(
            jax.ShapeDtypeStruct((1, T * B), jnp.float32),
            jax.ShapeDtypeStruct((1, B, H), jnp.float32),
            jax.ShapeDtypeStruct((1, B, H), jnp.float32),
        ),
        grid_spec=pltpu.PrefetchScalarGridSpec(
            num_scalar_prefetch=0,
            grid=(1,),
            in_specs=[
                pl.BlockSpec((T * B, d_in), lambda i: (0, 0)),
                pl.BlockSpec((1, B, H), lambda i: (0, 0, 0)),
                pl.BlockSpec((1, B, H), lambda i: (0, 0, 0)),
                pl.BlockSpec((L["epi_rows"], H), lambda i: (0, 0)),
                pl.BlockSpec(memory_space=pl.ANY),
            ],
            out_specs=[
                pl.BlockSpec((1, T * B), lambda i: (0, 0)),
                pl.BlockSpec((1, B, H), lambda i: (0, 0, 0)),
                pl.BlockSpec((1, B, H), lambda i: (0, 0, 0)),
            ],
            scratch_shapes=[
                pltpu.VMEM((L["o_wih"], H), jnp.float32),
                pltpu.VMEM((L["main_rows"] - L["o_wih"], G), jnp.float32),
                pltpu.SemaphoreType.DMA,
                pltpu.SemaphoreType.DMA,
            ],
        ),
        compiler_params=pltpu.CompilerParams(
            dimension_semantics=("arbitrary",),
        ),
        cost_estimate=cost,
    )(x, h0, c0, epi, main)

    q = q_row.reshape(T, B).T[..., None]
    return q, (h_n, c_n)
